# baseline (speedup 1.0000x reference)
import functools

import jax
import jax.numpy as jnp
from jax import lax
from jax.experimental import pallas as pl
from jax.experimental.pallas import tpu as pltpu
from jax.experimental.pallas import tpu_sc as plsc

N_EXPERTS = 8
HIDDEN = 2048
TOKENS = 8192
TBLK = 256
NT = TOKENS // TBLK
NPB = NT + N_EXPERTS
NP = NPB * TBLK

NWORKERS = 32
TOK_PER_W = TOKENS // NWORKERS
CH = 16
NCH = TOK_PER_W // CH
WPAD = 128


def _gating_kernel(x_ref, gw_ref, idx_ref, rank_ref, wgt_ref, cnt_ref, base_ref, run_ref):
    t = pl.program_id(0)

    @pl.when(t == 0)
    def _init():
        run_ref[...] = jnp.zeros_like(run_ref)

    x = x_ref[...]
    logits = lax.dot_general(x, gw_ref[...], (((1,), (1,)), ((), ())),
                             preferred_element_type=jnp.float32)
    m = jnp.max(logits, axis=1, keepdims=True)
    p = jnp.exp(logits - m)
    wgt_ref[...] = 1.0 / jnp.sum(p, axis=1)
    lane = lax.broadcasted_iota(jnp.int32, logits.shape, 1)
    idx = jnp.min(jnp.where(logits == m, lane, N_EXPERTS), axis=1)
    idx_ref[...] = idx

    lane16 = lax.broadcasted_iota(jnp.int32, (TBLK, 16), 1)
    onehot = (idx[:, None] == lane16).astype(jnp.float32)
    r = lax.broadcasted_iota(jnp.int32, (TBLK, TBLK), 0)
    c = lax.broadcasted_iota(jnp.int32, (TBLK, TBLK), 1)
    tri = (c < r).astype(jnp.float32)
    prefix = lax.dot_general(tri, onehot, (((1,), (0,)), ((), ())),
                             preferred_element_type=jnp.float32)
    run0 = run_ref[...].astype(jnp.float32)
    rank_ref[...] = jnp.sum((prefix + run0) * onehot, axis=1).astype(jnp.int32)
    run_ref[...] += jnp.sum(onehot, axis=0, keepdims=True).astype(jnp.int32)
    cnt_ref[...] = run_ref[...]
    nb = jnp.floor((run_ref[...].astype(jnp.float32) + (TBLK - 1)) * (1.0 / TBLK))
    ri = lax.broadcasted_iota(jnp.int32, (16, 16), 0)
    ci = lax.broadcasted_iota(jnp.int32, (16, 16), 1)
    upper = (ri < ci).astype(jnp.float32)
    base16 = lax.dot_general(nb, upper, (((1,), (0,)), ((), ())),
                             preferred_element_type=jnp.float32) * TBLK
    base_ref[...] = base16.astype(jnp.int32)


def _gating(x, gate_w):
    return pl.pallas_call(
        _gating_kernel,
        grid=(NT,),
        in_specs=[
            pl.BlockSpec((TBLK, HIDDEN), lambda t: (t, 0)),
            pl.BlockSpec((N_EXPERTS, HIDDEN), lambda t: (0, 0)),
        ],
        out_specs=[
            pl.BlockSpec((TBLK,), lambda t: (t,)),
            pl.BlockSpec((TBLK,), lambda t: (t,)),
            pl.BlockSpec((TBLK,), lambda t: (t,)),
            pl.BlockSpec((1, 16), lambda t: (0, 0)),
            pl.BlockSpec((1, 16), lambda t: (0, 0)),
        ],
        out_shape=[
            jax.ShapeDtypeStruct((TOKENS,), jnp.int32),
            jax.ShapeDtypeStruct((TOKENS,), jnp.int32),
            jax.ShapeDtypeStruct((TOKENS,), jnp.float32),
            jax.ShapeDtypeStruct((1, 16), jnp.int32),
            jax.ShapeDtypeStruct((1, 16), jnp.int32),
        ],
        scratch_shapes=[pltpu.VMEM((1, 16), jnp.int32)],
        compiler_params=pltpu.CompilerParams(
            dimension_semantics=("arbitrary",)),
    )(x, gate_w)


def _pos_kernel(idx_ref, rank_ref, wgt_ref, base_ref, pos_ref, w16_ref):
    idx = idx_ref[...]
    lane16 = lax.broadcasted_iota(jnp.int32, (TBLK, 16), 1)
    onehot = (idx[:, None] == lane16).astype(jnp.float32)
    base = base_ref[...].astype(jnp.float32)
    pos = jnp.sum(onehot * base, axis=1).astype(jnp.int32) + rank_ref[...]
    pos_ref[...] = pos
    w16_ref[...] = jnp.broadcast_to(wgt_ref[...][:, None], (TBLK, WPAD))


def _pos_weights(idx, rank, wgt, base16):
    return pl.pallas_call(
        _pos_kernel,
        grid=(NT,),
        in_specs=[
            pl.BlockSpec((TBLK,), lambda t: (t,)),
            pl.BlockSpec((TBLK,), lambda t: (t,)),
            pl.BlockSpec((TBLK,), lambda t: (t,)),
            pl.BlockSpec((1, 16), lambda t: (0, 0)),
        ],
        out_specs=[
            pl.BlockSpec((TBLK,), lambda t: (t,)),
            pl.BlockSpec((TBLK, WPAD), lambda t: (t, 0)),
        ],
        out_shape=[
            jax.ShapeDtypeStruct((TOKENS,), jnp.int32),
            jax.ShapeDtypeStruct((TOKENS, WPAD), jnp.float32),
        ],
        compiler_params=pltpu.CompilerParams(
            dimension_semantics=("arbitrary",)),
    )(idx, rank, wgt, base16)


def _dispatch_body(x_hbm, pos_hbm, w16_hbm,
                   xpad_hbm, wpad_hbm,
                   pos_v, w16_v, rows_v, sem_r, sem_x, sem_w):
    wid = lax.axis_index("s") * 2 + lax.axis_index("c")
    tok0 = wid * TOK_PER_W
    sx_h = [None, None]
    sw_h = [None, None]
    for ci in range(NCH):
        sl = ci % 2
        off = tok0 + ci * CH
        if ci >= 2:
            sx_h[sl].wait()
            sw_h[sl].wait()
        pltpu.sync_copy(pos_hbm.at[pl.ds(off, CH)], pos_v.at[sl])
        pltpu.sync_copy(w16_hbm.at[pl.ds(off, CH)], w16_v.at[sl])
        r_h = pltpu.async_copy(x_hbm.at[pl.ds(off, CH)], rows_v.at[sl], sem_r.at[sl])
        r_h.wait()
        sx_h[sl] = pltpu.async_copy(rows_v.at[sl], xpad_hbm.at[pos_v.at[sl]], sem_x.at[sl])
        sw_h[sl] = pltpu.async_copy(w16_v.at[sl], wpad_hbm.at[pos_v.at[sl]], sem_w.at[sl])
    for sl in range(2):
        sx_h[sl].wait()
        sw_h[sl].wait()


def _dispatch(x, pos, wgt16):
    mesh = plsc.VectorSubcoreMesh(core_axis_name="c", subcore_axis_name="s")
    fn = pl.kernel(
        _dispatch_body,
        out_type=[
            jax.ShapeDtypeStruct((NP, HIDDEN), jnp.float32),
            jax.ShapeDtypeStruct((NP, WPAD), jnp.float32),
        ],
        mesh=mesh,
        scratch_types=[
            pltpu.VMEM((2, CH), jnp.int32),
            pltpu.VMEM((2, CH, WPAD), jnp.float32),
            pltpu.VMEM((2, CH, HIDDEN), jnp.float32),
            pltpu.SemaphoreType.DMA((2,)),
            pltpu.SemaphoreType.DMA((2,)),
            pltpu.SemaphoreType.DMA((2,)),
        ],
    )
    return fn(x, pos, wgt16)


def _mm_kernel(bexp_ref, bval_ref, x_ref, w_ref, b_ref, wgt_ref, out_ref):
    i = pl.program_id(0)

    @pl.when(bval_ref[i] != 0)
    def _compute():
        xb = x_ref[...].astype(jnp.bfloat16)
        wb = w_ref[0].astype(jnp.bfloat16)
        y = lax.dot_general(xb, wb, (((1,), (1,)), ((), ())),
                            preferred_element_type=jnp.float32)
        y = y + b_ref[0]
        out_ref[...] = y * wgt_ref[:, :1]


def _grouped_matmul(x_pad, expert_w, expert_b, wgt_pad, bexp, bval):
    grid_spec = pltpu.PrefetchScalarGridSpec(
        num_scalar_prefetch=2,
        grid=(NPB,),
        in_specs=[
            pl.BlockSpec((TBLK, HIDDEN), lambda i, be, bv: (i, 0)),
            pl.BlockSpec((1, HIDDEN, HIDDEN), lambda i, be, bv: (be[i], 0, 0)),
            pl.BlockSpec((1, 1, HIDDEN), lambda i, be, bv: (be[i], 0, 0)),
            pl.BlockSpec((TBLK, WPAD), lambda i, be, bv: (i, 0)),
        ],
        out_specs=pl.BlockSpec((TBLK, HIDDEN), lambda i, be, bv: (i, 0)),
    )
    return pl.pallas_call(
        _mm_kernel,
        grid_spec=grid_spec,
        out_shape=jax.ShapeDtypeStruct((NP, HIDDEN), jnp.float32),
        compiler_params=pltpu.CompilerParams(
            dimension_semantics=("arbitrary",)),
    )(bexp, bval, x_pad, expert_w,
      expert_b.reshape(N_EXPERTS, 1, HIDDEN), wgt_pad)


def _combine_body(opad_hbm, pos_hbm, out_hbm, pos_v, rows_v, sem_g, sem_w):
    wid = lax.axis_index("s") * 2 + lax.axis_index("c")
    tok0 = wid * TOK_PER_W
    w_h = [None, None]
    for ci in range(NCH):
        sl = ci % 2
        off = tok0 + ci * CH
        if ci >= 2:
            w_h[sl].wait()
        pltpu.sync_copy(pos_hbm.at[pl.ds(off, CH)], pos_v.at[sl])
        g_h = pltpu.async_copy(opad_hbm.at[pos_v.at[sl]], rows_v.at[sl], sem_g.at[sl])
        g_h.wait()
        w_h[sl] = pltpu.async_copy(rows_v.at[sl], out_hbm.at[pl.ds(off, CH)], sem_w.at[sl])
    for sl in range(2):
        w_h[sl].wait()


def _combine(out_pad, pos):
    mesh = plsc.VectorSubcoreMesh(core_axis_name="c", subcore_axis_name="s")
    fn = pl.kernel(
        _combine_body,
        out_type=jax.ShapeDtypeStruct((TOKENS, HIDDEN), jnp.float32),
        mesh=mesh,
        scratch_types=[
            pltpu.VMEM((2, CH), jnp.int32),
            pltpu.VMEM((2, CH, HIDDEN), jnp.float32),
            pltpu.SemaphoreType.DMA((2,)),
            pltpu.SemaphoreType.DMA((2,)),
        ],
    )
    return fn(out_pad, pos)


def _routing_meta(counts):
    nb = (counts + TBLK - 1) // TBLK
    ends = jnp.cumsum(nb)
    total = ends[N_EXPERTS - 1]
    bid = jnp.arange(NPB, dtype=jnp.int32)
    bexp_raw = jnp.searchsorted(ends, bid, side="right").astype(jnp.int32)
    bval = (bid < total).astype(jnp.int32)
    last_e = jnp.searchsorted(ends, total - 1, side="right").astype(jnp.int32)
    bexp = jnp.where(bval != 0, jnp.minimum(bexp_raw, N_EXPERTS - 1), last_e)
    return bexp, bval


@jax.jit
def kernel(hidden_states, gate_w, expert_w, expert_b):
    b, s, h = hidden_states.shape
    x = hidden_states.reshape(-1, h)
    idx, rank, wgt, counts, base16 = _gating(x, gate_w)
    counts = counts[0, :N_EXPERTS]
    bexp, bval = _routing_meta(counts)
    pos, wgt16 = _pos_weights(idx, rank, wgt, base16)
    x_pad, wgt_pad = _dispatch(x, pos, wgt16)
    out_pad = _grouped_matmul(x_pad, expert_w, expert_b, wgt_pad, bexp, bval)
    out = _combine(out_pad, pos)
    return out.reshape(b, s, h)


# speedup vs baseline: 1.0365x; 1.0365x over previous
import functools

import jax
import jax.numpy as jnp
from jax import lax
from jax.experimental import pallas as pl
from jax.experimental.pallas import tpu as pltpu
from jax.experimental.pallas import tpu_sc as plsc

N_EXPERTS = 8
HIDDEN = 2048
TOKENS = 8192
TBLK = 512
NT = TOKENS // TBLK
NPB = NT + N_EXPERTS
NP = NPB * TBLK

NWORKERS = 32
TOK_PER_W = TOKENS // NWORKERS
CH = 32
NCH = TOK_PER_W // CH
WPAD = 128


def _gating_kernel(x_ref, gw_ref, idx_ref, rank_ref, wgt_ref, cnt_ref, base_ref,
                   bexp_ref, bval_ref, run_ref):
    t = pl.program_id(0)

    @pl.when(t == 0)
    def _init():
        run_ref[...] = jnp.zeros_like(run_ref)

    x = x_ref[...]
    logits = lax.dot_general(x, gw_ref[...], (((1,), (1,)), ((), ())),
                             preferred_element_type=jnp.float32)
    m = jnp.max(logits, axis=1, keepdims=True)
    p = jnp.exp(logits - m)
    wgt_ref[...] = 1.0 / jnp.sum(p, axis=1)
    lane = lax.broadcasted_iota(jnp.int32, logits.shape, 1)
    idx = jnp.min(jnp.where(logits == m, lane, N_EXPERTS), axis=1)
    idx_ref[...] = idx

    lane16 = lax.broadcasted_iota(jnp.int32, (TBLK, 16), 1)
    onehot = (idx[:, None] == lane16).astype(jnp.float32)
    r = lax.broadcasted_iota(jnp.int32, (TBLK, TBLK), 0)
    c = lax.broadcasted_iota(jnp.int32, (TBLK, TBLK), 1)
    tri = (c < r).astype(jnp.float32)
    prefix = lax.dot_general(tri, onehot, (((1,), (0,)), ((), ())),
                             preferred_element_type=jnp.float32)
    run0 = run_ref[...].astype(jnp.float32)
    rank_ref[...] = jnp.sum((prefix + run0) * onehot, axis=1).astype(jnp.int32)
    run_ref[...] += jnp.sum(onehot, axis=0, keepdims=True).astype(jnp.int32)
    cnt_ref[...] = run_ref[...]
    nb = jnp.floor((run_ref[...].astype(jnp.float32) + (TBLK - 1)) * (1.0 / TBLK))
    ri = lax.broadcasted_iota(jnp.int32, (16, 16), 0)
    ci = lax.broadcasted_iota(jnp.int32, (16, 16), 1)
    upper = (ri < ci).astype(jnp.float32)
    base16 = lax.dot_general(nb, upper, (((1,), (0,)), ((), ())),
                             preferred_element_type=jnp.float32) * TBLK
    base_ref[...] = base16.astype(jnp.int32)
    ends = base16 * (1.0 / TBLK) + nb
    total = jnp.sum(nb, axis=1, keepdims=True)
    bid = lax.broadcasted_iota(jnp.int32, (NPB, 16), 0).astype(jnp.float32)
    lane_ok = lax.broadcasted_iota(jnp.int32, (NPB, 16), 1) < N_EXPERTS
    cmp = jnp.where((ends <= bid) & lane_ok, 1.0, 0.0)
    bexp_raw = jnp.sum(cmp, axis=1)
    bval = (lax.broadcasted_iota(jnp.int32, (NPB,), 0).astype(jnp.float32) < total[0, 0])
    cmp_last = jnp.where((ends <= total[0, 0] - 1.0) & lane_ok, 1.0, 0.0)
    last_e = jnp.sum(cmp_last, axis=1)
    bexp = jnp.where(bval, jnp.minimum(bexp_raw, N_EXPERTS - 1.0), last_e)
    bexp_ref[...] = bexp.astype(jnp.int32)
    bval_ref[...] = bval.astype(jnp.int32)


def _gating(x, gate_w):
    return pl.pallas_call(
        _gating_kernel,
        grid=(NT,),
        in_specs=[
            pl.BlockSpec((TBLK, HIDDEN), lambda t: (t, 0)),
            pl.BlockSpec((N_EXPERTS, HIDDEN), lambda t: (0, 0)),
        ],
        out_specs=[
            pl.BlockSpec((TBLK,), lambda t: (t,)),
            pl.BlockSpec((TBLK,), lambda t: (t,)),
            pl.BlockSpec((TBLK,), lambda t: (t,)),
            pl.BlockSpec((1, 16), lambda t: (0, 0)),
            pl.BlockSpec((1, 16), lambda t: (0, 0)),
            pl.BlockSpec((NPB,), lambda t: (0,)),
            pl.BlockSpec((NPB,), lambda t: (0,)),
        ],
        out_shape=[
            jax.ShapeDtypeStruct((TOKENS,), jnp.int32),
            jax.ShapeDtypeStruct((TOKENS,), jnp.int32),
            jax.ShapeDtypeStruct((TOKENS,), jnp.float32),
            jax.ShapeDtypeStruct((1, 16), jnp.int32),
            jax.ShapeDtypeStruct((1, 16), jnp.int32),
            jax.ShapeDtypeStruct((NPB,), jnp.int32),
            jax.ShapeDtypeStruct((NPB,), jnp.int32),
        ],
        scratch_shapes=[pltpu.VMEM((1, 16), jnp.int32)],
        compiler_params=pltpu.CompilerParams(
            dimension_semantics=("arbitrary",)),
    )(x, gate_w)


def _pos_kernel(idx_ref, rank_ref, wgt_ref, base_ref, pos_ref, w16_ref):
    idx = idx_ref[...]
    lane16 = lax.broadcasted_iota(jnp.int32, (TBLK, 16), 1)
    onehot = (idx[:, None] == lane16).astype(jnp.float32)
    base = base_ref[...].astype(jnp.float32)
    pos = jnp.sum(onehot * base, axis=1).astype(jnp.int32) + rank_ref[...]
    pos_ref[...] = pos
    w16_ref[...] = jnp.broadcast_to(wgt_ref[...][:, None], (TBLK, WPAD))


def _pos_weights(idx, rank, wgt, base16):
    return pl.pallas_call(
        _pos_kernel,
        grid=(NT,),
        in_specs=[
            pl.BlockSpec((TBLK,), lambda t: (t,)),
            pl.BlockSpec((TBLK,), lambda t: (t,)),
            pl.BlockSpec((TBLK,), lambda t: (t,)),
            pl.BlockSpec((1, 16), lambda t: (0, 0)),
        ],
        out_specs=[
            pl.BlockSpec((TBLK,), lambda t: (t,)),
            pl.BlockSpec((TBLK, WPAD), lambda t: (t, 0)),
        ],
        out_shape=[
            jax.ShapeDtypeStruct((TOKENS,), jnp.int32),
            jax.ShapeDtypeStruct((TOKENS, WPAD), jnp.float32),
        ],
        compiler_params=pltpu.CompilerParams(
            dimension_semantics=("arbitrary",)),
    )(idx, rank, wgt, base16)


def _dispatch_body(x_hbm, pos_hbm, w16_hbm,
                   xpad_hbm, wpad_hbm,
                   pos_v, w16_v, rows_v, sem, sem2):
    wid = lax.axis_index("s") * 2 + lax.axis_index("c")
    tok0 = wid * TOK_PER_W
    for ci in range(NCH):
        off = tok0 + ci * CH
        pltpu.sync_copy(pos_hbm.at[pl.ds(off, CH)], pos_v)
        pltpu.sync_copy(w16_hbm.at[pl.ds(off, CH)], w16_v)
        pltpu.sync_copy(x_hbm.at[pl.ds(off, CH)], rows_v)
        pltpu.async_copy(rows_v, xpad_hbm.at[pos_v], sem).wait()
        pltpu.async_copy(w16_v, wpad_hbm.at[pos_v], sem2).wait()


def _dispatch(x, pos, wgt16):
    mesh = plsc.VectorSubcoreMesh(core_axis_name="c", subcore_axis_name="s")
    fn = pl.kernel(
        _dispatch_body,
        out_type=[
            jax.ShapeDtypeStruct((NP, HIDDEN), jnp.float32),
            jax.ShapeDtypeStruct((NP, WPAD), jnp.float32),
        ],
        mesh=mesh,
        scratch_types=[
            pltpu.VMEM((CH,), jnp.int32),
            pltpu.VMEM((CH, WPAD), jnp.float32),
            pltpu.VMEM((CH, HIDDEN), jnp.float32),
            pltpu.SemaphoreType.DMA,
            pltpu.SemaphoreType.DMA,
        ],
    )
    return fn(x, pos, wgt16)


def _mm_kernel(bexp_ref, bval_ref, x_ref, w_ref, b_ref, wgt_ref, out_ref):
    i = pl.program_id(0)

    @pl.when(bval_ref[i] != 0)
    def _compute():
        xb = x_ref[...].astype(jnp.bfloat16)
        wb = w_ref[0].astype(jnp.bfloat16)
        y = lax.dot_general(xb, wb, (((1,), (1,)), ((), ())),
                            preferred_element_type=jnp.float32)
        y = y + b_ref[0]
        out_ref[...] = y * wgt_ref[:, :1]


def _grouped_matmul(x_pad, expert_w, expert_b, wgt_pad, bexp, bval):
    grid_spec = pltpu.PrefetchScalarGridSpec(
        num_scalar_prefetch=2,
        grid=(NPB,),
        in_specs=[
            pl.BlockSpec((TBLK, HIDDEN), lambda i, be, bv: (i, 0)),
            pl.BlockSpec((1, HIDDEN, HIDDEN), lambda i, be, bv: (be[i], 0, 0)),
            pl.BlockSpec((1, 1, HIDDEN), lambda i, be, bv: (be[i], 0, 0)),
            pl.BlockSpec((TBLK, WPAD), lambda i, be, bv: (i, 0)),
        ],
        out_specs=pl.BlockSpec((TBLK, HIDDEN), lambda i, be, bv: (i, 0)),
    )
    return pl.pallas_call(
        _mm_kernel,
        grid_spec=grid_spec,
        out_shape=jax.ShapeDtypeStruct((NP, HIDDEN), jnp.float32),
        compiler_params=pltpu.CompilerParams(
            dimension_semantics=("arbitrary",)),
    )(bexp, bval, x_pad, expert_w,
      expert_b.reshape(N_EXPERTS, 1, HIDDEN), wgt_pad)


def _combine_body(opad_hbm, pos_hbm, out_hbm, pos_v, rows_v, sem):
    wid = lax.axis_index("s") * 2 + lax.axis_index("c")
    tok0 = wid * TOK_PER_W
    for ci in range(NCH):
        off = tok0 + ci * CH
        pltpu.sync_copy(pos_hbm.at[pl.ds(off, CH)], pos_v)
        pltpu.async_copy(opad_hbm.at[pos_v], rows_v, sem).wait()
        pltpu.sync_copy(rows_v, out_hbm.at[pl.ds(off, CH)])


def _combine(out_pad, pos):
    mesh = plsc.VectorSubcoreMesh(core_axis_name="c", subcore_axis_name="s")
    fn = pl.kernel(
        _combine_body,
        out_type=jax.ShapeDtypeStruct((TOKENS, HIDDEN), jnp.float32),
        mesh=mesh,
        scratch_types=[
            pltpu.VMEM((CH,), jnp.int32),
            pltpu.VMEM((CH, HIDDEN), jnp.float32),
            pltpu.SemaphoreType.DMA,
        ],
    )
    return fn(out_pad, pos)


@jax.jit
def kernel(hidden_states, gate_w, expert_w, expert_b):
    b, s, h = hidden_states.shape
    x = hidden_states.reshape(-1, h)
    idx, rank, wgt, _counts, base16, bexp, bval = _gating(x, gate_w)
    pos, wgt16 = _pos_weights(idx, rank, wgt, base16)
    x_pad, wgt_pad = _dispatch(x, pos, wgt16)
    out_pad = _grouped_matmul(x_pad, expert_w, expert_b, wgt_pad, bexp, bval)
    out = _combine(out_pad, pos)
    return out.reshape(b, s, h)


# speedup vs baseline: 1.0578x; 1.0205x over previous
import functools

import jax
import jax.numpy as jnp
from jax import lax
from jax.experimental import pallas as pl
from jax.experimental.pallas import tpu as pltpu
from jax.experimental.pallas import tpu_sc as plsc

N_EXPERTS = 8
HIDDEN = 2048
TOKENS = 8192
TBLK = 512
NT = TOKENS // TBLK
NPB = NT + N_EXPERTS
NP = NPB * TBLK

NWORKERS = 32
TOK_PER_W = TOKENS // NWORKERS
CH = 32
NCH = TOK_PER_W // CH
WPAD = 128


def _gating_kernel(x_ref, gw_ref, idx_ref, rank_ref, wgt_ref, cnt_ref, base_ref,
                   bexp_ref, bval_ref, run_ref):
    t = pl.program_id(0)

    @pl.when(t == 0)
    def _init():
        run_ref[...] = jnp.zeros_like(run_ref)

    x = x_ref[...]
    logits = lax.dot_general(x, gw_ref[...], (((1,), (1,)), ((), ())),
                             preferred_element_type=jnp.float32)
    m = jnp.max(logits, axis=1, keepdims=True)
    p = jnp.exp(logits - m)
    wgt_ref[...] = 1.0 / jnp.sum(p, axis=1)
    lane = lax.broadcasted_iota(jnp.int32, logits.shape, 1)
    idx = jnp.min(jnp.where(logits == m, lane, N_EXPERTS), axis=1)
    idx_ref[...] = idx

    lane16 = lax.broadcasted_iota(jnp.int32, (TBLK, 16), 1)
    onehot = (idx[:, None] == lane16).astype(jnp.float32)
    r = lax.broadcasted_iota(jnp.int32, (TBLK, TBLK), 0)
    c = lax.broadcasted_iota(jnp.int32, (TBLK, TBLK), 1)
    tri = (c < r).astype(jnp.float32)
    prefix = lax.dot_general(tri, onehot, (((1,), (0,)), ((), ())),
                             preferred_element_type=jnp.float32)
    run0 = run_ref[...].astype(jnp.float32)
    rank_ref[...] = jnp.sum((prefix + run0) * onehot, axis=1).astype(jnp.int32)
    run_ref[...] += jnp.sum(onehot, axis=0, keepdims=True).astype(jnp.int32)
    cnt_ref[...] = run_ref[...]
    nb = jnp.floor((run_ref[...].astype(jnp.float32) + (TBLK - 1)) * (1.0 / TBLK))
    ri = lax.broadcasted_iota(jnp.int32, (16, 16), 0)
    ci = lax.broadcasted_iota(jnp.int32, (16, 16), 1)
    upper = (ri < ci).astype(jnp.float32)
    base16 = lax.dot_general(nb, upper, (((1,), (0,)), ((), ())),
                             preferred_element_type=jnp.float32) * TBLK
    base_ref[...] = base16.astype(jnp.int32)
    ends = base16 * (1.0 / TBLK) + nb
    total = jnp.sum(nb, axis=1, keepdims=True)
    bid = lax.broadcasted_iota(jnp.int32, (NPB, 16), 0).astype(jnp.float32)
    lane_ok = lax.broadcasted_iota(jnp.int32, (NPB, 16), 1) < N_EXPERTS
    cmp = jnp.where((ends <= bid) & lane_ok, 1.0, 0.0)
    bexp_raw = jnp.sum(cmp, axis=1)
    bval = (lax.broadcasted_iota(jnp.int32, (NPB,), 0).astype(jnp.float32) < total[0, 0])
    cmp_last = jnp.where((ends <= total[0, 0] - 1.0) & lane_ok, 1.0, 0.0)
    last_e = jnp.sum(cmp_last, axis=1)
    bexp = jnp.where(bval, jnp.minimum(bexp_raw, N_EXPERTS - 1.0), last_e)
    bexp_ref[...] = bexp.astype(jnp.int32)
    bval_ref[...] = bval.astype(jnp.int32)


def _gating(x, gate_w):
    return pl.pallas_call(
        _gating_kernel,
        grid=(NT,),
        in_specs=[
            pl.BlockSpec((TBLK, HIDDEN), lambda t: (t, 0)),
            pl.BlockSpec((N_EXPERTS, HIDDEN), lambda t: (0, 0)),
        ],
        out_specs=[
            pl.BlockSpec((TBLK,), lambda t: (t,)),
            pl.BlockSpec((TBLK,), lambda t: (t,)),
            pl.BlockSpec((TBLK,), lambda t: (t,)),
            pl.BlockSpec((1, 16), lambda t: (0, 0)),
            pl.BlockSpec((1, 16), lambda t: (0, 0)),
            pl.BlockSpec((NPB,), lambda t: (0,)),
            pl.BlockSpec((NPB,), lambda t: (0,)),
        ],
        out_shape=[
            jax.ShapeDtypeStruct((TOKENS,), jnp.int32),
            jax.ShapeDtypeStruct((TOKENS,), jnp.int32),
            jax.ShapeDtypeStruct((TOKENS,), jnp.float32),
            jax.ShapeDtypeStruct((1, 16), jnp.int32),
            jax.ShapeDtypeStruct((1, 16), jnp.int32),
            jax.ShapeDtypeStruct((NPB,), jnp.int32),
            jax.ShapeDtypeStruct((NPB,), jnp.int32),
        ],
        scratch_shapes=[pltpu.VMEM((1, 16), jnp.int32)],
        compiler_params=pltpu.CompilerParams(
            dimension_semantics=("arbitrary",)),
    )(x, gate_w)


def _pos_kernel(idx_ref, rank_ref, wgt_ref, base_ref, pos_ref, w16_ref):
    idx = idx_ref[...]
    lane16 = lax.broadcasted_iota(jnp.int32, (TBLK, 16), 1)
    onehot = (idx[:, None] == lane16).astype(jnp.float32)
    base = base_ref[...].astype(jnp.float32)
    pos = jnp.sum(onehot * base, axis=1).astype(jnp.int32) + rank_ref[...]
    pos_ref[...] = pos
    w16_ref[...] = jnp.broadcast_to(wgt_ref[...][:, None], (TBLK, WPAD))


def _pos_weights(idx, rank, wgt, base16):
    return pl.pallas_call(
        _pos_kernel,
        grid=(NT,),
        in_specs=[
            pl.BlockSpec((TBLK,), lambda t: (t,)),
            pl.BlockSpec((TBLK,), lambda t: (t,)),
            pl.BlockSpec((TBLK,), lambda t: (t,)),
            pl.BlockSpec((1, 16), lambda t: (0, 0)),
        ],
        out_specs=[
            pl.BlockSpec((TBLK,), lambda t: (t,)),
            pl.BlockSpec((TBLK, WPAD), lambda t: (t, 0)),
        ],
        out_shape=[
            jax.ShapeDtypeStruct((TOKENS,), jnp.int32),
            jax.ShapeDtypeStruct((TOKENS, WPAD), jnp.float32),
        ],
        compiler_params=pltpu.CompilerParams(
            dimension_semantics=("arbitrary",)),
    )(idx, rank, wgt, base16)


def _dispatch_body(x_hbm, pos_hbm, w16_hbm,
                   xpad_hbm, wpad_hbm,
                   pos_v, w16_v, rows_v, sem, sem2):
    wid = lax.axis_index("s") * 2 + lax.axis_index("c")
    tok0 = wid * TOK_PER_W
    for ci in range(NCH):
        off = tok0 + ci * CH
        r_h = pltpu.async_copy(x_hbm.at[pl.ds(off, CH)], rows_v, sem)
        pltpu.sync_copy(pos_hbm.at[pl.ds(off, CH)], pos_v)
        pltpu.sync_copy(w16_hbm.at[pl.ds(off, CH)], w16_v)
        r_h.wait()
        sx_h = pltpu.async_copy(rows_v, xpad_hbm.at[pos_v], sem)
        sw_h = pltpu.async_copy(w16_v, wpad_hbm.at[pos_v], sem2)
        sx_h.wait()
        sw_h.wait()


def _dispatch(x, pos, wgt16):
    mesh = plsc.VectorSubcoreMesh(core_axis_name="c", subcore_axis_name="s")
    fn = pl.kernel(
        _dispatch_body,
        out_type=[
            jax.ShapeDtypeStruct((NP, HIDDEN), jnp.float32),
            jax.ShapeDtypeStruct((NP, WPAD), jnp.float32),
        ],
        mesh=mesh,
        scratch_types=[
            pltpu.VMEM((CH,), jnp.int32),
            pltpu.VMEM((CH, WPAD), jnp.float32),
            pltpu.VMEM((CH, HIDDEN), jnp.float32),
            pltpu.SemaphoreType.DMA,
            pltpu.SemaphoreType.DMA,
        ],
    )
    return fn(x, pos, wgt16)


def _mm_kernel(bexp_ref, bval_ref, x_ref, w_ref, b_ref, wgt_ref, out_ref):
    i = pl.program_id(0)

    @pl.when(bval_ref[i] != 0)
    def _compute():
        xb = x_ref[...].astype(jnp.bfloat16)
        wb = w_ref[0].astype(jnp.bfloat16)
        y = lax.dot_general(xb, wb, (((1,), (1,)), ((), ())),
                            preferred_element_type=jnp.float32)
        y = y + b_ref[0]
        out_ref[...] = y * wgt_ref[:, :1]


def _grouped_matmul(x_pad, expert_w, expert_b, wgt_pad, bexp, bval):
    grid_spec = pltpu.PrefetchScalarGridSpec(
        num_scalar_prefetch=2,
        grid=(NPB,),
        in_specs=[
            pl.BlockSpec((TBLK, HIDDEN), lambda i, be, bv: (i, 0)),
            pl.BlockSpec((1, HIDDEN, HIDDEN), lambda i, be, bv: (be[i], 0, 0)),
            pl.BlockSpec((1, 1, HIDDEN), lambda i, be, bv: (be[i], 0, 0)),
            pl.BlockSpec((TBLK, WPAD), lambda i, be, bv: (i, 0)),
        ],
        out_specs=pl.BlockSpec((TBLK, HIDDEN), lambda i, be, bv: (i, 0)),
    )
    return pl.pallas_call(
        _mm_kernel,
        grid_spec=grid_spec,
        out_shape=jax.ShapeDtypeStruct((NP, HIDDEN), jnp.float32),
        compiler_params=pltpu.CompilerParams(
            dimension_semantics=("arbitrary",)),
    )(bexp, bval, x_pad, expert_w,
      expert_b.reshape(N_EXPERTS, 1, HIDDEN), wgt_pad)


def _combine_body(opad_hbm, pos_hbm, out_hbm, pos_v, rows_v, sem):
    wid = lax.axis_index("s") * 2 + lax.axis_index("c")
    tok0 = wid * TOK_PER_W
    for ci in range(NCH):
        off = tok0 + ci * CH
        pltpu.sync_copy(pos_hbm.at[pl.ds(off, CH)], pos_v)
        pltpu.async_copy(opad_hbm.at[pos_v], rows_v, sem).wait()
        pltpu.sync_copy(rows_v, out_hbm.at[pl.ds(off, CH)])


def _combine(out_pad, pos):
    mesh = plsc.VectorSubcoreMesh(core_axis_name="c", subcore_axis_name="s")
    fn = pl.kernel(
        _combine_body,
        out_type=jax.ShapeDtypeStruct((TOKENS, HIDDEN), jnp.float32),
        mesh=mesh,
        scratch_types=[
            pltpu.VMEM((CH,), jnp.int32),
            pltpu.VMEM((CH, HIDDEN), jnp.float32),
            pltpu.SemaphoreType.DMA,
        ],
    )
    return fn(out_pad, pos)


@jax.jit
def kernel(hidden_states, gate_w, expert_w, expert_b):
    b, s, h = hidden_states.shape
    x = hidden_states.reshape(-1, h)
    idx, rank, wgt, _counts, base16, bexp, bval = _gating(x, gate_w)
    pos, wgt16 = _pos_weights(idx, rank, wgt, base16)
    x_pad, wgt_pad = _dispatch(x, pos, wgt16)
    out_pad = _grouped_matmul(x_pad, expert_w, expert_b, wgt_pad, bexp, bval)
    out = _combine(out_pad, pos)
    return out.reshape(b, s, h)


# speedup vs baseline: 1.0635x; 1.0054x over previous
import jax
import jax.numpy as jnp
from jax import lax
from jax.experimental import pallas as pl
from jax.experimental.pallas import tpu as pltpu
from jax.experimental.pallas import tpu_sc as plsc

N_EXPERTS = 8
HIDDEN = 2048
TOKENS = 8192
TBLK = 512
NT = TOKENS // TBLK
NPB = NT + N_EXPERTS
NP = NPB * TBLK

NWORKERS = 32
TOK_PER_W = TOKENS // NWORKERS
CH = 32
NCH = TOK_PER_W // CH
WPAD = 128


def _gating_kernel(x_ref, gw_ref, bexp_ref, bval_ref, pos_ref, w16_ref,
                   run_ref, base_s, idx_s, rank_s, wgt_s):
    t = pl.program_id(0)

    @pl.when(t == 0)
    def _init():
        run_ref[...] = jnp.zeros_like(run_ref)

    @pl.when(t < NT)
    def _phase_a():
        x = x_ref[...]
        logits = lax.dot_general(x, gw_ref[...], (((1,), (1,)), ((), ())),
                                 preferred_element_type=jnp.float32)
        m = jnp.max(logits, axis=1, keepdims=True)
        p = jnp.exp(logits - m)
        wgt_s[t] = 1.0 / jnp.sum(p, axis=1)
        lane = lax.broadcasted_iota(jnp.int32, logits.shape, 1)
        idx = jnp.min(jnp.where(logits == m, lane, N_EXPERTS), axis=1)
        idx_s[t] = idx

        lane16 = lax.broadcasted_iota(jnp.int32, (TBLK, 16), 1)
        onehot = (idx[:, None] == lane16).astype(jnp.float32)
        r = lax.broadcasted_iota(jnp.int32, (TBLK, TBLK), 0)
        c = lax.broadcasted_iota(jnp.int32, (TBLK, TBLK), 1)
        tri = (c < r).astype(jnp.float32)
        prefix = lax.dot_general(tri, onehot, (((1,), (0,)), ((), ())),
                                 preferred_element_type=jnp.float32)
        run0 = run_ref[...].astype(jnp.float32)
        rank_s[t] = jnp.sum((prefix + run0) * onehot, axis=1).astype(jnp.int32)
        run_ref[...] += jnp.sum(onehot, axis=0, keepdims=True).astype(jnp.int32)
        nb = jnp.floor((run_ref[...].astype(jnp.float32) + (TBLK - 1))
                       * (1.0 / TBLK))
        ri = lax.broadcasted_iota(jnp.int32, (16, 16), 0)
        ci = lax.broadcasted_iota(jnp.int32, (16, 16), 1)
        upper = (ri < ci).astype(jnp.float32)
        base16 = lax.dot_general(nb, upper, (((1,), (0,)), ((), ())),
                                 preferred_element_type=jnp.float32) * TBLK
        base_s[...] = base16.astype(jnp.int32)
        ends = base16 * (1.0 / TBLK) + nb
        total = jnp.sum(nb, axis=1, keepdims=True)
        bid = lax.broadcasted_iota(jnp.int32, (NPB, 16), 0).astype(jnp.float32)
        lane_ok = lax.broadcasted_iota(jnp.int32, (NPB, 16), 1) < N_EXPERTS
        cmp = jnp.where((ends <= bid) & lane_ok, 1.0, 0.0)
        bexp_raw = jnp.sum(cmp, axis=1)
        bval = (lax.broadcasted_iota(jnp.int32, (NPB,), 0).astype(jnp.float32)
                < total[0, 0])
        cmp_last = jnp.where((ends <= total[0, 0] - 1.0) & lane_ok, 1.0, 0.0)
        last_e = jnp.sum(cmp_last, axis=1)
        bexp = jnp.where(bval, jnp.minimum(bexp_raw, N_EXPERTS - 1.0), last_e)
        bexp_ref[...] = bexp.astype(jnp.int32)
        bval_ref[...] = bval.astype(jnp.int32)

    @pl.when(t >= NT)
    def _phase_b():
        u = t - NT
        idx = idx_s[u]
        lane16 = lax.broadcasted_iota(jnp.int32, (TBLK, 16), 1)
        onehot = (idx[:, None] == lane16).astype(jnp.float32)
        base = base_s[...].astype(jnp.float32)
        pos_ref[...] = (jnp.sum(onehot * base, axis=1).astype(jnp.int32)
                        + rank_s[u])
        w16_ref[...] = jnp.broadcast_to(wgt_s[u][:, None], (TBLK, WPAD))


def _gating(x, gate_w):
    return pl.pallas_call(
        _gating_kernel,
        grid=(2 * NT,),
        in_specs=[
            pl.BlockSpec((TBLK, HIDDEN), lambda t: (jnp.minimum(t, NT - 1), 0)),
            pl.BlockSpec((N_EXPERTS, HIDDEN), lambda t: (0, 0)),
        ],
        out_specs=[
            pl.BlockSpec((NPB,), lambda t: (0,)),
            pl.BlockSpec((NPB,), lambda t: (0,)),
            pl.BlockSpec((TBLK,), lambda t: (jnp.maximum(t - NT, 0),)),
            pl.BlockSpec((TBLK, WPAD), lambda t: (jnp.maximum(t - NT, 0), 0)),
        ],
        out_shape=[
            jax.ShapeDtypeStruct((NPB,), jnp.int32),
            jax.ShapeDtypeStruct((NPB,), jnp.int32),
            jax.ShapeDtypeStruct((TOKENS,), jnp.int32),
            jax.ShapeDtypeStruct((TOKENS, WPAD), jnp.float32),
        ],
        scratch_shapes=[
            pltpu.VMEM((1, 16), jnp.int32),
            pltpu.VMEM((1, 16), jnp.int32),
            pltpu.VMEM((NT, TBLK), jnp.int32),
            pltpu.VMEM((NT, TBLK), jnp.int32),
            pltpu.VMEM((NT, TBLK), jnp.float32),
        ],
        compiler_params=pltpu.CompilerParams(
            dimension_semantics=("arbitrary",)),
    )(x, gate_w)


def _dispatch_body(x_hbm, pos_hbm, w16_hbm,
                   xpad_hbm, wpad_hbm,
                   pos_v, w16_v, rows_v, sem, sem2):
    wid = lax.axis_index("s") * 2 + lax.axis_index("c")
    tok0 = wid * TOK_PER_W
    for ci in range(NCH):
        off = tok0 + ci * CH
        r_h = pltpu.async_copy(x_hbm.at[pl.ds(off, CH)], rows_v, sem)
        pltpu.sync_copy(pos_hbm.at[pl.ds(off, CH)], pos_v)
        pltpu.sync_copy(w16_hbm.at[pl.ds(off, CH)], w16_v)
        r_h.wait()
        sx_h = pltpu.async_copy(rows_v, xpad_hbm.at[pos_v], sem)
        sw_h = pltpu.async_copy(w16_v, wpad_hbm.at[pos_v], sem2)
        sx_h.wait()
        sw_h.wait()


def _dispatch(x, pos, wgt16):
    mesh = plsc.VectorSubcoreMesh(core_axis_name="c", subcore_axis_name="s")
    fn = pl.kernel(
        _dispatch_body,
        out_type=[
            jax.ShapeDtypeStruct((NP, HIDDEN), jnp.float32),
            jax.ShapeDtypeStruct((NP, WPAD), jnp.float32),
        ],
        mesh=mesh,
        scratch_types=[
            pltpu.VMEM((CH,), jnp.int32),
            pltpu.VMEM((CH, WPAD), jnp.float32),
            pltpu.VMEM((CH, HIDDEN), jnp.float32),
            pltpu.SemaphoreType.DMA,
            pltpu.SemaphoreType.DMA,
        ],
    )
    return fn(x, pos, wgt16)


def _mm_kernel(bexp_ref, bval_ref, x_ref, w_ref, b_ref, wgt_ref, out_ref):
    i = pl.program_id(0)

    @pl.when(bval_ref[i] != 0)
    def _compute():
        xb = x_ref[...].astype(jnp.bfloat16)
        wb = w_ref[0].astype(jnp.bfloat16)
        y = lax.dot_general(xb, wb, (((1,), (1,)), ((), ())),
                            preferred_element_type=jnp.float32)
        y = y + b_ref[0]
        out_ref[...] = y * wgt_ref[:, :1]


def _grouped_matmul(x_pad, expert_w, expert_b, wgt_pad, bexp, bval):
    grid_spec = pltpu.PrefetchScalarGridSpec(
        num_scalar_prefetch=2,
        grid=(NPB,),
        in_specs=[
            pl.BlockSpec((TBLK, HIDDEN), lambda i, be, bv: (i, 0)),
            pl.BlockSpec((1, HIDDEN, HIDDEN), lambda i, be, bv: (be[i], 0, 0)),
            pl.BlockSpec((1, 1, HIDDEN), lambda i, be, bv: (be[i], 0, 0)),
            pl.BlockSpec((TBLK, WPAD), lambda i, be, bv: (i, 0)),
        ],
        out_specs=pl.BlockSpec((TBLK, HIDDEN), lambda i, be, bv: (i, 0)),
    )
    return pl.pallas_call(
        _mm_kernel,
        grid_spec=grid_spec,
        out_shape=jax.ShapeDtypeStruct((NP, HIDDEN), jnp.float32),
        compiler_params=pltpu.CompilerParams(
            dimension_semantics=("arbitrary",)),
    )(bexp, bval, x_pad, expert_w,
      expert_b.reshape(N_EXPERTS, 1, HIDDEN), wgt_pad)


def _combine_body(opad_hbm, pos_hbm, out_hbm, pos_v, rows_v, sem):
    wid = lax.axis_index("s") * 2 + lax.axis_index("c")
    tok0 = wid * TOK_PER_W
    for ci in range(NCH):
        off = tok0 + ci * CH
        pltpu.sync_copy(pos_hbm.at[pl.ds(off, CH)], pos_v)
        pltpu.async_copy(opad_hbm.at[pos_v], rows_v, sem).wait()
        pltpu.sync_copy(rows_v, out_hbm.at[pl.ds(off, CH)])


def _combine(out_pad, pos):
    mesh = plsc.VectorSubcoreMesh(core_axis_name="c", subcore_axis_name="s")
    fn = pl.kernel(
        _combine_body,
        out_type=jax.ShapeDtypeStruct((TOKENS, HIDDEN), jnp.float32),
        mesh=mesh,
        scratch_types=[
            pltpu.VMEM((CH,), jnp.int32),
            pltpu.VMEM((CH, HIDDEN), jnp.float32),
            pltpu.SemaphoreType.DMA,
        ],
    )
    return fn(out_pad, pos)


@jax.jit
def kernel(hidden_states, gate_w, expert_w, expert_b):
    b, s, h = hidden_states.shape
    x = hidden_states.reshape(-1, h)
    bexp, bval, pos, wgt16 = _gating(x, gate_w)
    x_pad, wgt_pad = _dispatch(x, pos, wgt16)
    out_pad = _grouped_matmul(x_pad, expert_w, expert_b, wgt_pad, bexp, bval)
    out = _combine(out_pad, pos)
    return out.reshape(b, s, h)
